# Initial kernel scaffold; baseline (speedup 1.0000x reference)
import jax
import jax.numpy as jnp
from jax.experimental import pallas as pl


def kernel(x, edge_index, W, V, B):
    raise NotImplementedError("write your pallas kernel here")


# speedup vs baseline: 11.1634x; 11.1634x over previous
import functools

import jax
import jax.numpy as jnp
from jax import lax
from jax.experimental import pallas as pl
from jax.experimental.pallas import tpu as pltpu
from jax.experimental.pallas import tpu_sc as plsc

NC = 2
NS = 16
NW = NC * NS
CHUNK = 128


def _make_mesh():
    return plsc.VectorSubcoreMesh(core_axis_name="c", subcore_axis_name="s")


def _make_deg_kernel(nch, span, n_pad):
    @functools.partial(
        pl.kernel,
        out_type=jax.ShapeDtypeStruct((NC, NS, span), jnp.float32),
        mesh=_make_mesh(),
        scratch_types=[
            pltpu.VMEM((nch, CHUNK), jnp.int32),
            pltpu.VMEM((CHUNK,), jnp.float32),
            pltpu.VMEM((span,), jnp.float32),
            pltpu.VMEM_SHARED((n_pad,), jnp.float32),
        ],
    )
    def deg_kernel(row_hbm, deg_hbm, idx_v, ones_v, zero_v, deg_sh):
        cid = lax.axis_index("c")
        sid = lax.axis_index("s")
        wid = sid * NC + cid
        pltpu.sync_copy(row_hbm.at[wid], idx_v)

        def fill_ones(j, carry):
            ones_v[pl.ds(j * 16, 16)] = jnp.full((16,), 1.0, jnp.float32)
            return carry

        lax.fori_loop(0, CHUNK // 16, fill_ones, 0)

        def fill_zero(j, carry):
            zero_v[pl.ds(j * 16, 16)] = jnp.zeros((16,), jnp.float32)
            return carry

        lax.fori_loop(0, span // 16, fill_zero, 0)
        pltpu.sync_copy(zero_v, deg_sh.at[pl.ds(sid * span, span)])
        plsc.subcore_barrier()

        def body(j, carry):
            pltpu.sync_copy(ones_v, deg_sh.at[idx_v.at[j]], add=True)
            return carry

        lax.fori_loop(0, nch, body, 0)
        plsc.subcore_barrier()
        pltpu.sync_copy(deg_sh.at[pl.ds(sid * span, span)], deg_hbm.at[cid, sid])

    return deg_kernel


def _make_edge_kernel(nch, span, n_pad, f):
    @functools.partial(
        pl.kernel,
        out_type=jax.ShapeDtypeStruct((NC, NS, span, f), jnp.float32),
        mesh=_make_mesh(),
        scratch_types=[
            pltpu.VMEM((nch, CHUNK), jnp.int32),
            pltpu.VMEM((nch, CHUNK), jnp.int32),
            pltpu.VMEM((CHUNK, f), jnp.float32),
            pltpu.VMEM_SHARED((n_pad, f), jnp.float32),
            pltpu.SemaphoreType.DMA,
        ],
    )
    def edge_kernel(y_hbm, col_hbm, row_hbm, out_hbm,
                    col_v, row_v, buf, tmp_sh, sem0):
        cid = lax.axis_index("c")
        sid = lax.axis_index("s")
        wid = sid * NC + cid
        pltpu.sync_copy(col_hbm.at[wid], col_v)
        pltpu.sync_copy(row_hbm.at[wid], row_v)

        def fill_zero(t, carry):
            buf[t // (f // 16), pl.ds((t % (f // 16)) * 16, 16)] = (
                jnp.zeros((16,), jnp.float32))
            return carry

        lax.fori_loop(0, CHUNK * (f // 16), fill_zero, 0)
        for k in range(span // CHUNK):
            pltpu.sync_copy(
                buf, tmp_sh.at[pl.ds(sid * span + k * CHUNK, CHUNK)])
        plsc.subcore_barrier()

        def body(j, carry):
            pltpu.async_copy(y_hbm.at[col_v.at[j]], buf, sem0).wait()
            pltpu.sync_copy(buf, tmp_sh.at[row_v.at[j]], add=True)
            return carry

        lax.fori_loop(0, nch, body, 0)
        plsc.subcore_barrier()
        pltpu.sync_copy(tmp_sh.at[pl.ds(sid * span, span)], out_hbm.at[cid, sid])

    return edge_kernel


def _dense_body(x_ref, w_ref, v_ref, dp_ref, y_ref, skip_ref):
    d = dp_ref[0] + dp_ref[1]
    dinv = jnp.where(d > 0, lax.rsqrt(d), 0.0)
    xw = jnp.dot(x_ref[...], w_ref[...], preferred_element_type=jnp.float32)
    y_ref[...] = xw * dinv
    skip_ref[...] = jnp.dot(x_ref[...], v_ref[...], preferred_element_type=jnp.float32)


def _final_body(t_ref, dp_ref, skip_ref, b_ref, o_ref):
    d = dp_ref[0] + dp_ref[1]
    dinv = jnp.where(d > 0, lax.rsqrt(d), 0.0)
    agg = -(t_ref[0] + t_ref[1]) * dinv
    o_ref[...] = jnp.maximum(agg + skip_ref[...] + b_ref[...], 0.0)


def kernel(x, edge_index, W, V, B):
    n, f = x.shape
    e = edge_index.shape[1]

    per_dma = NW * CHUNK
    nch = -(-e // per_dma)
    nch += nch % 2
    e_pad = nch * per_dma
    span = -(-(n + 1) // NS)
    span = -(-span // CHUNK) * CHUNK
    n_pad = NS * span

    row = edge_index[0]
    col = edge_index[1]
    row_t = jnp.concatenate(
        [row, jnp.full((e_pad - e,), n, dtype=jnp.int32)]).reshape(NW, nch, CHUNK)
    col_t = jnp.concatenate(
        [col, jnp.zeros((e_pad - e,), dtype=jnp.int32)]).reshape(NW, nch, CHUNK)

    deg_p = _make_deg_kernel(nch, span, n_pad)(row_t)
    deg_p3 = deg_p.reshape(NC, n_pad, 1)

    bn = 2000
    grid = (n // bn,)
    y, skip = pl.pallas_call(
        _dense_body,
        grid=grid,
        in_specs=[
            pl.BlockSpec((bn, f), lambda i: (i, 0)),
            pl.BlockSpec((f, f), lambda i: (0, 0)),
            pl.BlockSpec((f, f), lambda i: (0, 0)),
            pl.BlockSpec((NC, bn, 1), lambda i: (0, i, 0)),
        ],
        out_specs=[pl.BlockSpec((bn, f), lambda i: (i, 0))] * 2,
        out_shape=[jax.ShapeDtypeStruct((n, f), jnp.float32)] * 2,
    )(x, W[0], V[0], deg_p3)

    tmp = _make_edge_kernel(nch, span, n_pad, f)(y, col_t, row_t)
    tmp = tmp.reshape(NC, n_pad, f)

    out = pl.pallas_call(
        _final_body,
        grid=grid,
        in_specs=[
            pl.BlockSpec((NC, bn, f), lambda i: (0, i, 0)),
            pl.BlockSpec((NC, bn, 1), lambda i: (0, i, 0)),
            pl.BlockSpec((bn, f), lambda i: (i, 0)),
            pl.BlockSpec((1, f), lambda i: (0, 0)),
        ],
        out_specs=pl.BlockSpec((bn, f), lambda i: (i, 0)),
        out_shape=jax.ShapeDtypeStruct((n, f), jnp.float32),
    )(tmp, deg_p3, skip, B[0])
    return out


# speedup vs baseline: 13.0624x; 1.1701x over previous
import functools

import jax
import jax.numpy as jnp
from jax import lax
from jax.experimental import pallas as pl
from jax.experimental.pallas import tpu as pltpu
from jax.experimental.pallas import tpu_sc as plsc

NC = 2
NS = 16
NW = NC * NS
CHUNK = 128


def _make_mesh():
    return plsc.VectorSubcoreMesh(core_axis_name="c", subcore_axis_name="s")


def _make_deg_kernel(nch, span, n_pad):
    @functools.partial(
        pl.kernel,
        out_type=jax.ShapeDtypeStruct((NC, NS, span), jnp.float32),
        mesh=_make_mesh(),
        scratch_types=[
            pltpu.VMEM((nch, CHUNK), jnp.int32),
            pltpu.VMEM((CHUNK,), jnp.float32),
            pltpu.VMEM((span,), jnp.float32),
            pltpu.VMEM_SHARED((n_pad,), jnp.float32),
            pltpu.SemaphoreType.DMA,
        ],
    )
    def deg_kernel(row_hbm, deg_hbm, idx_v, ones_v, zero_v, deg_sh, sem0):
        cid = lax.axis_index("c")
        sid = lax.axis_index("s")
        wid = sid * NC + cid
        pltpu.sync_copy(row_hbm.at[wid], idx_v)

        def fill_ones(j, carry):
            ones_v[pl.ds(j * 16, 16)] = jnp.full((16,), 1.0, jnp.float32)
            return carry

        lax.fori_loop(0, CHUNK // 16, fill_ones, 0)

        def fill_zero(j, carry):
            zero_v[pl.ds(j * 16, 16)] = jnp.zeros((16,), jnp.float32)
            return carry

        lax.fori_loop(0, span // 16, fill_zero, 0)
        pltpu.sync_copy(zero_v, deg_sh.at[pl.ds(sid * span, span)])
        plsc.subcore_barrier()

        def body(j, carry):
            pltpu.async_copy(ones_v, deg_sh.at[idx_v.at[j]], sem0, add=True)
            return carry

        lax.fori_loop(0, nch, body, 0)

        def drain(j, carry):
            pltpu.make_async_copy(ones_v, deg_sh.at[idx_v.at[j]], sem0).wait()
            return carry

        lax.fori_loop(0, nch, drain, 0)
        plsc.subcore_barrier()
        pltpu.sync_copy(deg_sh.at[pl.ds(sid * span, span)], deg_hbm.at[cid, sid])

    return deg_kernel


def _make_edge_kernel(nch, span, n_pad, f):
    @functools.partial(
        pl.kernel,
        out_type=jax.ShapeDtypeStruct((NC, NS, span, f), jnp.float32),
        mesh=_make_mesh(),
        scratch_types=[
            pltpu.VMEM((nch, CHUNK), jnp.int32),
            pltpu.VMEM((2, CHUNK), jnp.int32),
            pltpu.VMEM((2, CHUNK, f), jnp.float32),
            pltpu.VMEM_SHARED((n_pad, f), jnp.float32),
            pltpu.SemaphoreType.DMA,
            pltpu.SemaphoreType.DMA,
            pltpu.SemaphoreType.DMA,
            pltpu.SemaphoreType.DMA,
        ],
    )
    def edge_kernel(y_hbm, col_hbm, row_hbm, out_hbm,
                    col_v, row_v, buf, tmp_sh, semg0, semg1, semr0, semr1):
        cid = lax.axis_index("c")
        sid = lax.axis_index("s")
        wid = sid * NC + cid
        pltpu.sync_copy(col_hbm.at[wid], col_v)

        def fill_zero(t, carry):
            buf[0, t // (f // 16), pl.ds((t % (f // 16)) * 16, 16)] = (
                jnp.zeros((16,), jnp.float32))
            return carry

        lax.fori_loop(0, CHUNK * (f // 16), fill_zero, 0)
        for k in range(span // CHUNK):
            pltpu.sync_copy(
                buf.at[0], tmp_sh.at[pl.ds(sid * span + k * CHUNK, CHUNK)])
        plsc.subcore_barrier()

        pltpu.async_copy(row_hbm.at[wid, 0], row_v.at[0], semr0)
        pltpu.async_copy(y_hbm.at[col_v.at[0]], buf.at[0], semg0)

        def body(g, carry):
            base = g * 2
            pltpu.async_copy(row_hbm.at[wid, base + 1], row_v.at[1], semr1)
            pltpu.async_copy(y_hbm.at[col_v.at[base + 1]], buf.at[1], semg1)
            pltpu.make_async_copy(row_hbm.at[wid, base], row_v.at[0], semr0).wait()
            pltpu.make_async_copy(y_hbm.at[col_v.at[base]], buf.at[0], semg0).wait()
            pltpu.sync_copy(buf.at[0], tmp_sh.at[row_v.at[0]], add=True)

            @pl.when(base + 2 < nch)
            def _():
                pltpu.async_copy(row_hbm.at[wid, base + 2], row_v.at[0], semr0)
                pltpu.async_copy(y_hbm.at[col_v.at[base + 2]], buf.at[0], semg0)

            pltpu.make_async_copy(row_hbm.at[wid, base + 1], row_v.at[1], semr1).wait()
            pltpu.make_async_copy(y_hbm.at[col_v.at[base + 1]], buf.at[1], semg1).wait()
            pltpu.sync_copy(buf.at[1], tmp_sh.at[row_v.at[1]], add=True)
            return carry

        lax.fori_loop(0, nch // 2, body, 0)
        plsc.subcore_barrier()
        pltpu.sync_copy(tmp_sh.at[pl.ds(sid * span, span)], out_hbm.at[cid, sid])

    return edge_kernel


def _dense_body(x_ref, w_ref, v_ref, dp_ref, y_ref, skip_ref):
    d = dp_ref[0] + dp_ref[1]
    dinv = jnp.where(d > 0, lax.rsqrt(d), 0.0)
    xw = jnp.dot(x_ref[...], w_ref[...], preferred_element_type=jnp.float32)
    y_ref[...] = xw * dinv
    skip_ref[...] = jnp.dot(x_ref[...], v_ref[...], preferred_element_type=jnp.float32)


def _final_body(t_ref, dp_ref, skip_ref, b_ref, o_ref):
    d = dp_ref[0] + dp_ref[1]
    dinv = jnp.where(d > 0, lax.rsqrt(d), 0.0)
    agg = -(t_ref[0] + t_ref[1]) * dinv
    o_ref[...] = jnp.maximum(agg + skip_ref[...] + b_ref[...], 0.0)


def kernel(x, edge_index, W, V, B):
    n, f = x.shape
    e = edge_index.shape[1]

    per_dma = NW * CHUNK
    nch = -(-e // per_dma)
    nch += nch % 2
    e_pad = nch * per_dma
    span = -(-(n + 1) // NS)
    span = -(-span // CHUNK) * CHUNK
    n_pad = NS * span

    row = edge_index[0]
    col = edge_index[1]
    row_t = jnp.concatenate(
        [row, jnp.full((e_pad - e,), n, dtype=jnp.int32)]).reshape(NW, nch, CHUNK)
    col_t = jnp.concatenate(
        [col, jnp.zeros((e_pad - e,), dtype=jnp.int32)]).reshape(NW, nch, CHUNK)

    deg_p = _make_deg_kernel(nch, span, n_pad)(row_t)
    deg_p3 = deg_p.reshape(NC, n_pad, 1)

    bn = 2000
    grid = (n // bn,)
    y, skip = pl.pallas_call(
        _dense_body,
        grid=grid,
        in_specs=[
            pl.BlockSpec((bn, f), lambda i: (i, 0)),
            pl.BlockSpec((f, f), lambda i: (0, 0)),
            pl.BlockSpec((f, f), lambda i: (0, 0)),
            pl.BlockSpec((NC, bn, 1), lambda i: (0, i, 0)),
        ],
        out_specs=[pl.BlockSpec((bn, f), lambda i: (i, 0))] * 2,
        out_shape=[jax.ShapeDtypeStruct((n, f), jnp.float32)] * 2,
    )(x, W[0], V[0], deg_p3)

    tmp = _make_edge_kernel(nch, span, n_pad, f)(y, col_t, row_t)
    tmp = tmp.reshape(NC, n_pad, f)

    out = pl.pallas_call(
        _final_body,
        grid=grid,
        in_specs=[
            pl.BlockSpec((NC, bn, f), lambda i: (0, i, 0)),
            pl.BlockSpec((NC, bn, 1), lambda i: (0, i, 0)),
            pl.BlockSpec((bn, f), lambda i: (i, 0)),
            pl.BlockSpec((1, f), lambda i: (0, 0)),
        ],
        out_specs=pl.BlockSpec((bn, f), lambda i: (i, 0)),
        out_shape=jax.ShapeDtypeStruct((n, f), jnp.float32),
    )(tmp, deg_p3, skip, B[0])
    return out
